# Initial kernel scaffold; baseline (speedup 1.0000x reference)
import jax
import jax.numpy as jnp
from jax.experimental import pallas as pl


def kernel(h, pos, edge_index, params):
    raise NotImplementedError("write your pallas kernel here")


# speedup vs baseline: 1.7038x; 1.7038x over previous
import functools

import jax
import jax.numpy as jnp
from jax import lax
from jax.experimental import pallas as pl
from jax.experimental.pallas import tpu as pltpu
from jax.experimental.pallas import tpu_sc as plsc

NC = 2
NS = 16
F32 = jnp.float32


def _sc_gather(tbl, pos4, src, dst):
    n = tbl.shape[0]
    e = src.shape[0]
    nw = NC * NS
    per_w = e // nw
    ch = 512
    sub = 128
    iters = per_w // ch
    mesh = plsc.VectorSubcoreMesh(core_axis_name="c", subcore_axis_name="s")

    @functools.partial(
        pl.kernel,
        out_type=[jax.ShapeDtypeStruct((e, 128), F32),
                  jax.ShapeDtypeStruct((e, 128), F32),
                  jax.ShapeDtypeStruct((e * 4,), F32)],
        mesh=mesh,
        scratch_types=[
            pltpu.VMEM((ch,), jnp.int32),
            pltpu.VMEM((ch,), jnp.int32),
            pltpu.VMEM((ch, 128), F32),
            pltpu.VMEM((ch * 4,), F32),
            pltpu.VMEM((n * 4,), F32),
            pltpu.SemaphoreType.DMA,
        ],
        compiler_params=pltpu.CompilerParams(needs_layout_passes=False),
    )
    def k(t_hbm, p_hbm, src_hbm, dst_hbm, ts_hbm, td_hbm, dx_hbm,
          si, di, rbuf, dbuf, posv, sem):
        wid = lax.axis_index("s") * NC + lax.axis_index("c")
        base = wid * per_w
        lane = lax.iota(jnp.int32, 16)
        pltpu.sync_copy(p_hbm, posv)
        zv = jnp.zeros((16,), F32)
        for q in range(ch * 4 // 16):
            dbuf[pl.ds(q * 16, 16)] = zv

        def body_a(i, carry):
            off = base + i * ch
            pltpu.sync_copy(src_hbm.at[pl.ds(off, ch)], si)
            pltpu.sync_copy(dst_hbm.at[pl.ds(off, ch)], di)
            for j in range(ch // sub):
                pltpu.async_copy(
                    t_hbm.at[si.at[pl.ds(j * sub, sub)]],
                    rbuf.at[pl.ds(j * sub, sub)], sem)
            for g in range(ch // 16):
                s16 = si[pl.ds(g * 16, 16)]
                d16 = di[pl.ds(g * 16, 16)]
                row4 = (lane + g * 16) * 4
                for c in range(3):
                    cv = jnp.full((16,), c, jnp.int32)
                    xs = plsc.load_gather(posv, [s16 * 4 + cv])
                    xd = plsc.load_gather(posv, [d16 * 4 + cv])
                    plsc.store_scatter(dbuf, [row4 + cv], xs - xd)
            for j in range(ch // sub):
                pltpu.make_async_copy(
                    t_hbm.at[si.at[pl.ds(j * sub, sub)]],
                    rbuf.at[pl.ds(j * sub, sub)], sem).wait()
            pltpu.sync_copy(rbuf, ts_hbm.at[pl.ds(off, ch)])
            pltpu.sync_copy(dbuf, dx_hbm.at[pl.ds(off * 4, ch * 4)])
            return carry

        def body_b(i, carry):
            off = base + i * ch
            pltpu.sync_copy(dst_hbm.at[pl.ds(off, ch)], di)
            for j in range(ch // sub):
                pltpu.async_copy(
                    t_hbm.at[di.at[pl.ds(j * sub, sub)]],
                    rbuf.at[pl.ds(j * sub, sub)], sem)
            for j in range(ch // sub):
                pltpu.make_async_copy(
                    t_hbm.at[di.at[pl.ds(j * sub, sub)]],
                    rbuf.at[pl.ds(j * sub, sub)], sem).wait()
            pltpu.sync_copy(rbuf, td_hbm.at[pl.ds(off, ch)])
            return carry

        lax.fori_loop(0, iters, body_a, 0)
        lax.fori_loop(0, iters, body_b, 0)

    return k(tbl, pos4, src, dst)


def _sc_scatter(m0, m1, msgp, dst2d, nacc):
    sub = dst2d.shape[1]
    e = dst2d.shape[0] * sub
    ch = 640
    rpi = ch // sub
    per_t = e // NS
    iters = per_t // ch
    rows_pt = nacc // NS
    zr = 32
    mesh = plsc.VectorSubcoreMesh(core_axis_name="c", subcore_axis_name="s")

    @functools.partial(
        pl.kernel,
        out_type=[jax.ShapeDtypeStruct((nacc, 128), F32),
                  jax.ShapeDtypeStruct((nacc, 128), F32),
                  jax.ShapeDtypeStruct((nacc, 128), F32)],
        mesh=mesh,
        scratch_types=[
            pltpu.VMEM((rpi, sub), jnp.int32),
            pltpu.VMEM((2 * sub, 128), F32),
            pltpu.VMEM((zr, 128), F32),
            pltpu.VMEM_SHARED((nacc, 128), F32),
        ],
        compiler_params=pltpu.CompilerParams(needs_layout_passes=False),
    )
    def k(m0_hbm, m1_hbm, msg_hbm, idx_hbm, g0_hbm, g1_hbm, gx_hbm,
          idxv, rbuf, zb, smem):
        cid = lax.axis_index("c")
        sid = lax.axis_index("s")
        zv = jnp.zeros((16,), F32)
        for i in range(zr):
            for j in range(8):
                zb[i, pl.ds(j * 16, 16)] = zv
        r0 = sid * rows_pt

        def zero_acc():
            for q in range(rows_pt // zr):
                pltpu.sync_copy(zb, smem.at[pl.ds(r0 + q * zr, zr)])

        def accumulate(src_hbm):
            def body(i, carry):
                off = sid * per_t + i * ch
                irow = pl.multiple_of(off // sub, 8)
                pltpu.sync_copy(idx_hbm.at[pl.ds(irow, rpi)], idxv)
                for q in range(rpi // 2):
                    pltpu.sync_copy(
                        src_hbm.at[pl.ds(off + q * 2 * sub, 2 * sub)], rbuf)
                    for j in range(2):
                        pltpu.sync_copy(rbuf.at[pl.ds(j * sub, sub)],
                                        smem.at[idxv.at[q * 2 + j]], add=True)
                return carry
            lax.fori_loop(0, iters, body, 0)

        def writeout(out_hbm):
            pltpu.sync_copy(smem.at[pl.ds(r0, rows_pt)],
                            out_hbm.at[pl.ds(r0, rows_pt)])

        @pl.when(cid == 0)
        def _():
            zero_acc()
            plsc.subcore_barrier()
            accumulate(m0_hbm)
            plsc.subcore_barrier()
            writeout(g0_hbm)
            plsc.subcore_barrier()
            zero_acc()
            plsc.subcore_barrier()
            accumulate(msg_hbm)
            plsc.subcore_barrier()
            writeout(gx_hbm)

        @pl.when(cid == 1)
        def _():
            zero_acc()
            plsc.subcore_barrier()
            accumulate(m1_hbm)
            plsc.subcore_barrier()
            writeout(g1_hbm)

    return k(m0, m1, msgp, dst2d)


def _silu(x):
    return x * jax.nn.sigmoid(x)


def _edge_body(ts_ref, td_ref, dx_ref, w1a, w1b, w1r, b1, w2, b2, wc1, bc1,
               wc2, m0_ref, m1_ref, msg_ref):
    hs = ts_ref[...]
    hd = td_ref[...]
    d = dx_ref[...]
    r = jnp.sum(d * d, axis=-1, keepdims=True)
    u = d / (jnp.sqrt(r) + 1e-30)
    a1 = (jnp.dot(hs, w1a[...], preferred_element_type=F32)
          + jnp.dot(hd, w1b[...], preferred_element_type=F32)
          + r * w1r[...] + b1[...])
    m1v = _silu(a1)
    a2 = jnp.dot(m1v, w2[...], preferred_element_type=F32) + b2[...]
    mv = _silu(a2)
    a3 = jnp.dot(mv, wc1[...], preferred_element_type=F32) + bc1[...]
    cv = _silu(a3)
    coef = jnp.sum(cv * wc2[...], axis=-1, keepdims=True)
    msg = coef * u
    m0_ref[...] = mv[:, :128]
    m1_ref[...] = mv[:, 128:]
    msg_ref[...] = jnp.concatenate(
        [msg, jnp.zeros((msg.shape[0], 124), F32)], axis=1)


def _tc_edge(ts, td, dx, wts):
    e = ts.shape[0]
    rb = 1280
    const = lambda i: (0, 0)
    row = lambda i: (i, 0)
    wspecs = [pl.BlockSpec(w.shape, const) for w in wts]
    return pl.pallas_call(
        _edge_body,
        grid=(e // rb,),
        in_specs=[pl.BlockSpec((rb, 128), row), pl.BlockSpec((rb, 128), row),
                  pl.BlockSpec((rb, 4), row)] + wspecs,
        out_specs=[pl.BlockSpec((rb, 128), row), pl.BlockSpec((rb, 128), row),
                   pl.BlockSpec((rb, 128), row)],
        out_shape=[jax.ShapeDtypeStruct((e, 128), F32),
                   jax.ShapeDtypeStruct((e, 128), F32),
                   jax.ShapeDtypeStruct((e, 128), F32)],
    )(ts, td, dx, *wts)


def _node_body(t_ref, x_ref, g0_ref, g1_ref, gx_ref, wn1a, wn1b0, wn1b1, bn1,
               wn2, bn2, h_out, x_out):
    a = (jnp.dot(t_ref[...], wn1a[...], preferred_element_type=F32)
         + jnp.dot(g0_ref[...], wn1b0[...], preferred_element_type=F32)
         + jnp.dot(g1_ref[...], wn1b1[...], preferred_element_type=F32)
         + bn1[...])
    av = _silu(a)
    h_out[...] = jnp.dot(av, wn2[...], preferred_element_type=F32) + bn2[...]
    x_out[...] = x_ref[...] + gx_ref[...][:, :4]


def _tc_node(tbl, pos4, g0, g1, gx, wts):
    n = tbl.shape[0]
    rb = 1000
    const = lambda i: (0, 0)
    row = lambda i: (i, 0)
    wspecs = [pl.BlockSpec(w.shape, const) for w in wts]
    return pl.pallas_call(
        _node_body,
        grid=(n // rb,),
        in_specs=[pl.BlockSpec((rb, 128), row), pl.BlockSpec((rb, 4), row),
                  pl.BlockSpec((rb, 128), row), pl.BlockSpec((rb, 128), row),
                  pl.BlockSpec((rb, 128), row)] + wspecs,
        out_specs=[pl.BlockSpec((rb, 128), row), pl.BlockSpec((rb, 4), row)],
        out_shape=[jax.ShapeDtypeStruct((n, 128), F32),
                   jax.ShapeDtypeStruct((n, 4), F32)],
    )(tbl, pos4, g0, g1, gx, *wts)


def _reduce_body(t_ref, x_ref, hs_ref, xs_ref):
    sh = jnp.sum(t_ref[...], axis=0, keepdims=True)
    sx = jnp.sum(x_ref[...], axis=0, keepdims=True)

    @pl.when(pl.program_id(0) == 0)
    def _():
        hs_ref[...] = sh
        xs_ref[...] = sx

    @pl.when(pl.program_id(0) != 0)
    def _():
        hs_ref[...] += sh
        xs_ref[...] += sx


def _tc_reduce(tbl, pos4):
    n = tbl.shape[0]
    rb = 1000
    return pl.pallas_call(
        _reduce_body,
        grid=(n // rb,),
        in_specs=[pl.BlockSpec((rb, 128), lambda i: (i, 0)),
                  pl.BlockSpec((rb, 4), lambda i: (i, 0))],
        out_specs=[pl.BlockSpec((1, 128), lambda i: (0, 0)),
                   pl.BlockSpec((1, 4), lambda i: (0, 0))],
        out_shape=[jax.ShapeDtypeStruct((1, 128), F32),
                   jax.ShapeDtypeStruct((1, 4), F32)],
    )(tbl, pos4)


def _layer_weights(p):
    e1w = p["edge1"]["w"]
    edge = (e1w[:128], e1w[128:256], e1w[256:257],
            p["edge1"]["b"].reshape(1, -1),
            p["edge2"]["w"], p["edge2"]["b"].reshape(1, -1),
            p["coord1"]["w"], p["coord1"]["b"].reshape(1, -1),
            p["coord2"]["w"].reshape(1, -1))
    n1w = p["node1"]["w"]
    node = (n1w[:128], n1w[128:256], n1w[256:384],
            p["node1"]["b"].reshape(1, -1),
            p["node2"]["w"], p["node2"]["b"].reshape(1, -1))
    return edge, node


def kernel(h, pos, edge_index, params):
    n = h.shape[0]
    e = edge_index.shape[1]
    ep = -(-e // 10240) * 10240
    nacc = -(-(n + 1) // 640) * 640
    src = jnp.pad(edge_index[0], (0, ep - e))
    dst_g = jnp.pad(edge_index[1], (0, ep - e))
    dst2d = jnp.pad(edge_index[1], (0, ep - e),
                    constant_values=n).reshape(-1, 80)
    tbl = h
    pos4 = jnp.pad(pos, ((0, 0), (0, 1)))
    for name in ("layer1", "layer2", "layer3"):
        ewts, nwts = _layer_weights(params[name])
        ts, td, dxf = _sc_gather(tbl, pos4.reshape(-1), src, dst_g)
        dx = dxf.reshape(-1, 4)
        m0, m1, msgp = _tc_edge(ts, td, dx, ewts)
        g0, g1, gx = _sc_scatter(m0, m1, msgp, dst2d, nacc)
        tbl, pos4 = _tc_node(tbl, pos4, g0, g1, gx, nwts)
    hsums, xsums = _tc_reduce(tbl, pos4)
    return jnp.concatenate([hsums, xsums[:, :3]], axis=1) / n


# speedup vs baseline: 1.7708x; 1.0393x over previous
import functools

import jax
import jax.numpy as jnp
from jax import lax
from jax.experimental import pallas as pl
from jax.experimental.pallas import tpu as pltpu
from jax.experimental.pallas import tpu_sc as plsc

NC = 2
NS = 16
F32 = jnp.float32


def _sc_gather(tbl, pos4, src, dst):
    n = tbl.shape[0]
    e = src.shape[0]
    nw = NC * NS
    per_w = e // nw
    ch = 128
    blk = 1024
    iters = per_w // ch
    outer = per_w // blk
    mesh = plsc.VectorSubcoreMesh(core_axis_name="c", subcore_axis_name="s")

    @functools.partial(
        pl.kernel,
        out_type=[jax.ShapeDtypeStruct((e, 128), F32),
                  jax.ShapeDtypeStruct((e, 128), F32),
                  jax.ShapeDtypeStruct((e * 4,), F32)],
        mesh=mesh,
        scratch_types=[
            pltpu.VMEM((blk,), jnp.int32),
            pltpu.VMEM((blk,), jnp.int32),
            pltpu.VMEM((2 * ch, 128), F32),
            pltpu.VMEM((2 * ch, 128), F32),
            pltpu.VMEM((2 * ch * 4,), F32),
            pltpu.VMEM((n * 4,), F32),
            pltpu.SemaphoreType.DMA,
            pltpu.SemaphoreType.DMA,
        ],
        compiler_params=pltpu.CompilerParams(needs_layout_passes=False),
    )
    def k(t_hbm, p_hbm, src_hbm, dst_hbm, ts_hbm, td_hbm, dx_hbm,
          si, di, rs, rd, dbuf, posv, sem_g, sem_w):
        wid = lax.axis_index("s") * NC + lax.axis_index("c")
        base = wid * per_w
        lane = lax.iota(jnp.int32, 16)
        pltpu.sync_copy(p_hbm, posv)
        zv = jnp.zeros((16,), F32)
        for q in range(2 * ch * 4 // 16):
            dbuf[pl.ds(q * 16, 16)] = zv

        def wait_wb(i, b):
            off = base + i * ch
            pltpu.make_async_copy(rs.at[pl.ds(b * ch, ch)],
                                  ts_hbm.at[pl.ds(off, ch)], sem_w).wait()
            pltpu.make_async_copy(rd.at[pl.ds(b * ch, ch)],
                                  td_hbm.at[pl.ds(off, ch)], sem_w).wait()
            pltpu.make_async_copy(dbuf.at[pl.ds(b * ch * 4, ch * 4)],
                                  dx_hbm.at[pl.ds(off * 4, ch * 4)],
                                  sem_w).wait()

        def step(i, u):
            b = u % 2
            off = base + i * ch

            @pl.when(i >= 2)
            def _():
                wait_wb(i - 2, b)

            s1 = si.at[pl.ds(u * ch, ch)]
            d1 = di.at[pl.ds(u * ch, ch)]
            pltpu.async_copy(t_hbm.at[s1], rs.at[pl.ds(b * ch, ch)], sem_g)
            pltpu.async_copy(t_hbm.at[d1], rd.at[pl.ds(b * ch, ch)], sem_g)
            for g in range(ch // 16):
                s16 = si[pl.ds(u * ch + g * 16, 16)]
                d16 = di[pl.ds(u * ch + g * 16, 16)]
                row4 = b * ch * 4 + (lane + g * 16) * 4
                for c in range(3):
                    cv = jnp.full((16,), c, jnp.int32)
                    xs = plsc.load_gather(posv, [s16 * 4 + cv])
                    xd = plsc.load_gather(posv, [d16 * 4 + cv])
                    plsc.store_scatter(dbuf, [row4 + cv], xs - xd)
            pltpu.make_async_copy(t_hbm.at[s1],
                                  rs.at[pl.ds(b * ch, ch)], sem_g).wait()
            pltpu.make_async_copy(t_hbm.at[d1],
                                  rd.at[pl.ds(b * ch, ch)], sem_g).wait()
            pltpu.async_copy(rs.at[pl.ds(b * ch, ch)],
                             ts_hbm.at[pl.ds(off, ch)], sem_w)
            pltpu.async_copy(rd.at[pl.ds(b * ch, ch)],
                             td_hbm.at[pl.ds(off, ch)], sem_w)
            pltpu.async_copy(dbuf.at[pl.ds(b * ch * 4, ch * 4)],
                             dx_hbm.at[pl.ds(off * 4, ch * 4)], sem_w)

        def outer_body(g, carry):
            boff = base + g * blk
            pltpu.sync_copy(src_hbm.at[pl.ds(boff, blk)], si)
            pltpu.sync_copy(dst_hbm.at[pl.ds(boff, blk)], di)
            for u in range(blk // ch):
                step(g * (blk // ch) + u, u)
            return carry

        lax.fori_loop(0, outer, outer_body, 0)
        wait_wb(iters - 2, (iters - 2) % 2)
        wait_wb(iters - 1, (iters - 1) % 2)

    return k(tbl, pos4, src, dst)


def _sc_scatter(m0, m1, msgp, dst2d, nacc):
    sub = dst2d.shape[1]
    e = dst2d.shape[0] * sub
    ch = 640
    rpi = ch // sub
    per_t = e // NS
    iters = per_t // ch
    rows_pt = nacc // NS
    zr = 32
    mesh = plsc.VectorSubcoreMesh(core_axis_name="c", subcore_axis_name="s")

    @functools.partial(
        pl.kernel,
        out_type=[jax.ShapeDtypeStruct((nacc, 128), F32),
                  jax.ShapeDtypeStruct((nacc, 128), F32),
                  jax.ShapeDtypeStruct((nacc, 128), F32)],
        mesh=mesh,
        scratch_types=[
            pltpu.VMEM((rpi, sub), jnp.int32),
            pltpu.VMEM((2 * sub, 128), F32),
            pltpu.VMEM((zr, 128), F32),
            pltpu.VMEM_SHARED((nacc, 128), F32),
        ],
        compiler_params=pltpu.CompilerParams(needs_layout_passes=False),
    )
    def k(m0_hbm, m1_hbm, msg_hbm, idx_hbm, g0_hbm, g1_hbm, gx_hbm,
          idxv, rbuf, zb, smem):
        cid = lax.axis_index("c")
        sid = lax.axis_index("s")
        zv = jnp.zeros((16,), F32)
        for i in range(zr):
            for j in range(8):
                zb[i, pl.ds(j * 16, 16)] = zv
        r0 = sid * rows_pt

        def zero_acc():
            for q in range(rows_pt // zr):
                pltpu.sync_copy(zb, smem.at[pl.ds(r0 + q * zr, zr)])

        def accumulate(src_hbm):
            def body(i, carry):
                off = sid * per_t + i * ch
                irow = pl.multiple_of(off // sub, 8)
                pltpu.sync_copy(idx_hbm.at[pl.ds(irow, rpi)], idxv)
                for q in range(rpi // 2):
                    pltpu.sync_copy(
                        src_hbm.at[pl.ds(off + q * 2 * sub, 2 * sub)], rbuf)
                    for j in range(2):
                        pltpu.sync_copy(rbuf.at[pl.ds(j * sub, sub)],
                                        smem.at[idxv.at[q * 2 + j]], add=True)
                return carry
            lax.fori_loop(0, iters, body, 0)

        def writeout(out_hbm):
            pltpu.sync_copy(smem.at[pl.ds(r0, rows_pt)],
                            out_hbm.at[pl.ds(r0, rows_pt)])

        @pl.when(cid == 0)
        def _():
            zero_acc()
            plsc.subcore_barrier()
            accumulate(m0_hbm)
            plsc.subcore_barrier()
            writeout(g0_hbm)
            plsc.subcore_barrier()
            zero_acc()
            plsc.subcore_barrier()
            accumulate(msg_hbm)
            plsc.subcore_barrier()
            writeout(gx_hbm)

        @pl.when(cid == 1)
        def _():
            zero_acc()
            plsc.subcore_barrier()
            accumulate(m1_hbm)
            plsc.subcore_barrier()
            writeout(g1_hbm)

    return k(m0, m1, msgp, dst2d)


def _silu(x):
    return x * jax.nn.sigmoid(x)


def _edge_body(ts_ref, td_ref, dx_ref, w1a, w1b, w1r, b1, w2, b2, wc1, bc1,
               wc2, m0_ref, m1_ref, msg_ref):
    hs = ts_ref[...]
    hd = td_ref[...]
    d = dx_ref[...]
    r = jnp.sum(d * d, axis=-1, keepdims=True)
    u = d / (jnp.sqrt(r) + 1e-30)
    a1 = (jnp.dot(hs, w1a[...], preferred_element_type=F32)
          + jnp.dot(hd, w1b[...], preferred_element_type=F32)
          + r * w1r[...] + b1[...])
    m1v = _silu(a1)
    a2 = jnp.dot(m1v, w2[...], preferred_element_type=F32) + b2[...]
    mv = _silu(a2)
    a3 = jnp.dot(mv, wc1[...], preferred_element_type=F32) + bc1[...]
    cv = _silu(a3)
    coef = jnp.sum(cv * wc2[...], axis=-1, keepdims=True)
    msg = coef * u
    m0_ref[...] = mv[:, :128]
    m1_ref[...] = mv[:, 128:]
    msg_ref[...] = jnp.concatenate(
        [msg, jnp.zeros((msg.shape[0], 124), F32)], axis=1)


def _tc_edge(ts, td, dx, wts):
    e = ts.shape[0]
    rb = 1280
    const = lambda i: (0, 0)
    row = lambda i: (i, 0)
    wspecs = [pl.BlockSpec(w.shape, const) for w in wts]
    return pl.pallas_call(
        _edge_body,
        grid=(e // rb,),
        in_specs=[pl.BlockSpec((rb, 128), row), pl.BlockSpec((rb, 128), row),
                  pl.BlockSpec((rb, 4), row)] + wspecs,
        out_specs=[pl.BlockSpec((rb, 128), row), pl.BlockSpec((rb, 128), row),
                   pl.BlockSpec((rb, 128), row)],
        out_shape=[jax.ShapeDtypeStruct((e, 128), F32),
                   jax.ShapeDtypeStruct((e, 128), F32),
                   jax.ShapeDtypeStruct((e, 128), F32)],
    )(ts, td, dx, *wts)


def _node_body(t_ref, x_ref, g0_ref, g1_ref, gx_ref, wn1a, wn1b0, wn1b1, bn1,
               wn2, bn2, h_out, x_out):
    a = (jnp.dot(t_ref[...], wn1a[...], preferred_element_type=F32)
         + jnp.dot(g0_ref[...], wn1b0[...], preferred_element_type=F32)
         + jnp.dot(g1_ref[...], wn1b1[...], preferred_element_type=F32)
         + bn1[...])
    av = _silu(a)
    h_out[...] = jnp.dot(av, wn2[...], preferred_element_type=F32) + bn2[...]
    x_out[...] = x_ref[...] + gx_ref[...][:, :4]


def _tc_node(tbl, pos4, g0, g1, gx, wts):
    n = tbl.shape[0]
    rb = 1000
    const = lambda i: (0, 0)
    row = lambda i: (i, 0)
    wspecs = [pl.BlockSpec(w.shape, const) for w in wts]
    return pl.pallas_call(
        _node_body,
        grid=(n // rb,),
        in_specs=[pl.BlockSpec((rb, 128), row), pl.BlockSpec((rb, 4), row),
                  pl.BlockSpec((rb, 128), row), pl.BlockSpec((rb, 128), row),
                  pl.BlockSpec((rb, 128), row)] + wspecs,
        out_specs=[pl.BlockSpec((rb, 128), row), pl.BlockSpec((rb, 4), row)],
        out_shape=[jax.ShapeDtypeStruct((n, 128), F32),
                   jax.ShapeDtypeStruct((n, 4), F32)],
    )(tbl, pos4, g0, g1, gx, *wts)


def _reduce_body(t_ref, x_ref, hs_ref, xs_ref):
    sh = jnp.sum(t_ref[...], axis=0, keepdims=True)
    sx = jnp.sum(x_ref[...], axis=0, keepdims=True)

    @pl.when(pl.program_id(0) == 0)
    def _():
        hs_ref[...] = sh
        xs_ref[...] = sx

    @pl.when(pl.program_id(0) != 0)
    def _():
        hs_ref[...] += sh
        xs_ref[...] += sx


def _tc_reduce(tbl, pos4):
    n = tbl.shape[0]
    rb = 1000
    return pl.pallas_call(
        _reduce_body,
        grid=(n // rb,),
        in_specs=[pl.BlockSpec((rb, 128), lambda i: (i, 0)),
                  pl.BlockSpec((rb, 4), lambda i: (i, 0))],
        out_specs=[pl.BlockSpec((1, 128), lambda i: (0, 0)),
                   pl.BlockSpec((1, 4), lambda i: (0, 0))],
        out_shape=[jax.ShapeDtypeStruct((1, 128), F32),
                   jax.ShapeDtypeStruct((1, 4), F32)],
    )(tbl, pos4)


def _layer_weights(p):
    e1w = p["edge1"]["w"]
    edge = (e1w[:128], e1w[128:256], e1w[256:257],
            p["edge1"]["b"].reshape(1, -1),
            p["edge2"]["w"], p["edge2"]["b"].reshape(1, -1),
            p["coord1"]["w"], p["coord1"]["b"].reshape(1, -1),
            p["coord2"]["w"].reshape(1, -1))
    n1w = p["node1"]["w"]
    node = (n1w[:128], n1w[128:256], n1w[256:384],
            p["node1"]["b"].reshape(1, -1),
            p["node2"]["w"], p["node2"]["b"].reshape(1, -1))
    return edge, node


def kernel(h, pos, edge_index, params):
    n = h.shape[0]
    e = edge_index.shape[1]
    ep = -(-e // 10240) * 10240
    nacc = -(-(n + 1) // 640) * 640
    src = jnp.pad(edge_index[0], (0, ep - e))
    dst_g = jnp.pad(edge_index[1], (0, ep - e))
    dst2d = jnp.pad(edge_index[1], (0, ep - e),
                    constant_values=n).reshape(-1, 80)
    tbl = h
    pos4 = jnp.pad(pos, ((0, 0), (0, 1)))
    for name in ("layer1", "layer2", "layer3"):
        ewts, nwts = _layer_weights(params[name])
        ts, td, dxf = _sc_gather(tbl, pos4.reshape(-1), src, dst_g)
        dx = dxf.reshape(-1, 4)
        m0, m1, msgp = _tc_edge(ts, td, dx, ewts)
        g0, g1, gx = _sc_scatter(m0, m1, msgp, dst2d, nacc)
        tbl, pos4 = _tc_node(tbl, pos4, g0, g1, gx, nwts)
    hsums, xsums = _tc_reduce(tbl, pos4)
    return jnp.concatenate([hsums, xsums[:, :3]], axis=1) / n


# speedup vs baseline: 3.0796x; 1.7391x over previous
import functools

import jax
import jax.numpy as jnp
from jax import lax
from jax.experimental import pallas as pl
from jax.experimental.pallas import tpu as pltpu
from jax.experimental.pallas import tpu_sc as plsc

NC = 2
NS = 16
F32 = jnp.float32


def _sc_dx(pos4, src, dst):
    n4 = pos4.shape[0]
    e = src.shape[0]
    nw = NC * NS
    per_w = e // nw
    ch = 512
    blk = 1024
    iters = per_w // ch
    outer = per_w // blk
    mesh = plsc.VectorSubcoreMesh(core_axis_name="c", subcore_axis_name="s")

    @functools.partial(
        pl.kernel,
        out_type=jax.ShapeDtypeStruct((e * 4,), F32),
        mesh=mesh,
        scratch_types=[
            pltpu.VMEM((blk,), jnp.int32),
            pltpu.VMEM((blk,), jnp.int32),
            pltpu.VMEM((2 * ch * 4,), F32),
            pltpu.VMEM((n4,), F32),
            pltpu.SemaphoreType.DMA,
        ],
        compiler_params=pltpu.CompilerParams(needs_layout_passes=False),
    )
    def k(p_hbm, src_hbm, dst_hbm, dx_hbm, si, di, dbuf, posv, sem_w):
        wid = lax.axis_index("s") * NC + lax.axis_index("c")
        base = wid * per_w
        lane = lax.iota(jnp.int32, 16)
        pltpu.sync_copy(p_hbm, posv)
        zv = jnp.zeros((16,), F32)
        for q in range(2 * ch * 4 // 16):
            dbuf[pl.ds(q * 16, 16)] = zv

        def wait_wb(i, b):
            off = base + i * ch
            pltpu.make_async_copy(dbuf.at[pl.ds(b * ch * 4, ch * 4)],
                                  dx_hbm.at[pl.ds(off * 4, ch * 4)],
                                  sem_w).wait()

        def step(i, u):
            b = u % 2
            off = base + i * ch

            @pl.when(i >= 2)
            def _():
                wait_wb(i - 2, b)

            for g in range(ch // 16):
                s16 = si[pl.ds(u * ch + g * 16, 16)]
                d16 = di[pl.ds(u * ch + g * 16, 16)]
                row4 = b * ch * 4 + (lane + g * 16) * 4
                for c in range(3):
                    cv = jnp.full((16,), c, jnp.int32)
                    xs = plsc.load_gather(posv, [s16 * 4 + cv])
                    xd = plsc.load_gather(posv, [d16 * 4 + cv])
                    plsc.store_scatter(dbuf, [row4 + cv], xs - xd)
            pltpu.async_copy(dbuf.at[pl.ds(b * ch * 4, ch * 4)],
                             dx_hbm.at[pl.ds(off * 4, ch * 4)], sem_w)

        def outer_body(g, carry):
            boff = base + g * blk
            pltpu.sync_copy(src_hbm.at[pl.ds(boff, blk)], si)
            pltpu.sync_copy(dst_hbm.at[pl.ds(boff, blk)], di)
            for u in range(blk // ch):
                step(g * (blk // ch) + u, u)
            return carry

        lax.fori_loop(0, outer, outer_body, 0)
        wait_wb(iters - 2, (iters - 2) % 2)
        wait_wb(iters - 1, (iters - 1) % 2)

    return k(pos4, src, dst)


def _sc_gather(tblp, src, dst):
    npad = tblp.shape[0]
    e = src.shape[0]
    nw = NC * NS
    per_w = e // nw
    ch = 80
    blk = 640
    iters = per_w // ch
    outer = per_w // blk
    rows_cp = npad // NS
    mesh = plsc.VectorSubcoreMesh(core_axis_name="c", subcore_axis_name="s")

    @functools.partial(
        pl.kernel,
        out_type=[jax.ShapeDtypeStruct((e, 128), F32),
                  jax.ShapeDtypeStruct((e, 128), F32)],
        mesh=mesh,
        scratch_types=[
            pltpu.VMEM((blk,), jnp.int32),
            pltpu.VMEM((blk,), jnp.int32),
            pltpu.VMEM((2 * ch, 128), F32),
            pltpu.VMEM((2 * ch, 128), F32),
            pltpu.VMEM_SHARED((npad, 128), F32),
            pltpu.SemaphoreType.DMA,
            pltpu.SemaphoreType.DMA,
        ],
        compiler_params=pltpu.CompilerParams(needs_layout_passes=False),
    )
    def k(t_hbm, src_hbm, dst_hbm, ts_hbm, td_hbm,
          si, di, rs, rd, t_sp, sem_g, sem_w):
        sid = lax.axis_index("s")
        wid = sid * NC + lax.axis_index("c")
        base = wid * per_w
        pltpu.sync_copy(t_hbm.at[pl.ds(sid * rows_cp, rows_cp)],
                        t_sp.at[pl.ds(sid * rows_cp, rows_cp)])
        plsc.subcore_barrier()

        def wait_wb(i, b):
            off = base + i * ch
            pltpu.make_async_copy(rs.at[pl.ds(b * ch, ch)],
                                  ts_hbm.at[pl.ds(off, ch)], sem_w).wait()
            pltpu.make_async_copy(rd.at[pl.ds(b * ch, ch)],
                                  td_hbm.at[pl.ds(off, ch)], sem_w).wait()

        def step(i, u):
            b = u % 2
            off = base + i * ch

            @pl.when(i >= 2)
            def _():
                wait_wb(i - 2, b)

            s1 = si.at[pl.ds(u * ch, ch)]
            d1 = di.at[pl.ds(u * ch, ch)]
            pltpu.async_copy(t_sp.at[s1], rs.at[pl.ds(b * ch, ch)], sem_g)
            pltpu.async_copy(t_sp.at[d1], rd.at[pl.ds(b * ch, ch)], sem_g)
            pltpu.make_async_copy(t_sp.at[s1],
                                  rs.at[pl.ds(b * ch, ch)], sem_g).wait()
            pltpu.make_async_copy(t_sp.at[d1],
                                  rd.at[pl.ds(b * ch, ch)], sem_g).wait()
            pltpu.async_copy(rs.at[pl.ds(b * ch, ch)],
                             ts_hbm.at[pl.ds(off, ch)], sem_w)
            pltpu.async_copy(rd.at[pl.ds(b * ch, ch)],
                             td_hbm.at[pl.ds(off, ch)], sem_w)

        def outer_body(g, carry):
            boff = base + g * blk
            pltpu.sync_copy(src_hbm.at[pl.ds(boff, blk)], si)
            pltpu.sync_copy(dst_hbm.at[pl.ds(boff, blk)], di)
            for u in range(blk // ch):
                step(g * (blk // ch) + u, u)
            return carry

        lax.fori_loop(0, outer, outer_body, 0)
        wait_wb(iters - 2, (iters - 2) % 2)
        wait_wb(iters - 1, (iters - 1) % 2)

    return k(tblp, src, dst)


def _sc_scatter(m0, m1, msgp, dst2d, nacc):
    sub = dst2d.shape[1]
    e = dst2d.shape[0] * sub
    ch = 640
    rpi = ch // sub
    per_t = e // NS
    iters = per_t // ch
    rows_pt = nacc // NS
    zr = 32
    mesh = plsc.VectorSubcoreMesh(core_axis_name="c", subcore_axis_name="s")

    @functools.partial(
        pl.kernel,
        out_type=[jax.ShapeDtypeStruct((nacc, 128), F32),
                  jax.ShapeDtypeStruct((nacc, 128), F32),
                  jax.ShapeDtypeStruct((nacc, 128), F32)],
        mesh=mesh,
        scratch_types=[
            pltpu.VMEM((rpi, sub), jnp.int32),
            pltpu.VMEM((2 * sub, 128), F32),
            pltpu.VMEM((zr, 128), F32),
            pltpu.VMEM_SHARED((nacc, 128), F32),
        ],
        compiler_params=pltpu.CompilerParams(needs_layout_passes=False),
    )
    def k(m0_hbm, m1_hbm, msg_hbm, idx_hbm, g0_hbm, g1_hbm, gx_hbm,
          idxv, rbuf, zb, smem):
        cid = lax.axis_index("c")
        sid = lax.axis_index("s")
        zv = jnp.zeros((16,), F32)
        for i in range(zr):
            for j in range(8):
                zb[i, pl.ds(j * 16, 16)] = zv
        r0 = sid * rows_pt

        def zero_acc():
            for q in range(rows_pt // zr):
                pltpu.sync_copy(zb, smem.at[pl.ds(r0 + q * zr, zr)])

        def accumulate(src_hbm):
            def body(i, carry):
                off = sid * per_t + i * ch
                irow = pl.multiple_of(off // sub, 8)
                pltpu.sync_copy(idx_hbm.at[pl.ds(irow, rpi)], idxv)
                for q in range(rpi // 2):
                    pltpu.sync_copy(
                        src_hbm.at[pl.ds(off + q * 2 * sub, 2 * sub)], rbuf)
                    for j in range(2):
                        pltpu.sync_copy(rbuf.at[pl.ds(j * sub, sub)],
                                        smem.at[idxv.at[q * 2 + j]], add=True)
                return carry
            lax.fori_loop(0, iters, body, 0)

        def writeout(out_hbm):
            pltpu.sync_copy(smem.at[pl.ds(r0, rows_pt)],
                            out_hbm.at[pl.ds(r0, rows_pt)])

        @pl.when(cid == 0)
        def _():
            zero_acc()
            plsc.subcore_barrier()
            accumulate(m0_hbm)
            plsc.subcore_barrier()
            writeout(g0_hbm)
            plsc.subcore_barrier()
            zero_acc()
            plsc.subcore_barrier()
            accumulate(msg_hbm)
            plsc.subcore_barrier()
            writeout(gx_hbm)

        @pl.when(cid == 1)
        def _():
            zero_acc()
            plsc.subcore_barrier()
            accumulate(m1_hbm)
            plsc.subcore_barrier()
            writeout(g1_hbm)

    return k(m0, m1, msgp, dst2d)


def _silu(x):
    return x * jax.nn.sigmoid(x)


def _edge_body(ts_ref, td_ref, dx_ref, w1a, w1b, w1r, b1, w2, b2, wc1, bc1,
               wc2, m0_ref, m1_ref, msg_ref):
    hs = ts_ref[...]
    hd = td_ref[...]
    d = dx_ref[...]
    r = jnp.sum(d * d, axis=-1, keepdims=True)
    u = d / (jnp.sqrt(r) + 1e-30)
    a1 = (jnp.dot(hs, w1a[...], preferred_element_type=F32)
          + jnp.dot(hd, w1b[...], preferred_element_type=F32)
          + r * w1r[...] + b1[...])
    m1v = _silu(a1)
    a2 = jnp.dot(m1v, w2[...], preferred_element_type=F32) + b2[...]
    mv = _silu(a2)
    a3 = jnp.dot(mv, wc1[...], preferred_element_type=F32) + bc1[...]
    cv = _silu(a3)
    coef = jnp.sum(cv * wc2[...], axis=-1, keepdims=True)
    msg = coef * u
    m0_ref[...] = mv[:, :128]
    m1_ref[...] = mv[:, 128:]
    msg_ref[...] = jnp.concatenate(
        [msg, jnp.zeros((msg.shape[0], 124), F32)], axis=1)


def _tc_edge(ts, td, dx, wts):
    e = ts.shape[0]
    rb = 1280
    const = lambda i: (0, 0)
    row = lambda i: (i, 0)
    wspecs = [pl.BlockSpec(w.shape, const) for w in wts]
    return pl.pallas_call(
        _edge_body,
        grid=(e // rb,),
        in_specs=[pl.BlockSpec((rb, 128), row), pl.BlockSpec((rb, 128), row),
                  pl.BlockSpec((rb, 4), row)] + wspecs,
        out_specs=[pl.BlockSpec((rb, 128), row), pl.BlockSpec((rb, 128), row),
                   pl.BlockSpec((rb, 128), row)],
        out_shape=[jax.ShapeDtypeStruct((e, 128), F32),
                   jax.ShapeDtypeStruct((e, 128), F32),
                   jax.ShapeDtypeStruct((e, 128), F32)],
    )(ts, td, dx, *wts)


def _node_body(t_ref, x_ref, g0_ref, g1_ref, gx_ref, wn1a, wn1b0, wn1b1, bn1,
               wn2, bn2, h_out, x_out):
    a = (jnp.dot(t_ref[...], wn1a[...], preferred_element_type=F32)
         + jnp.dot(g0_ref[...], wn1b0[...], preferred_element_type=F32)
         + jnp.dot(g1_ref[...], wn1b1[...], preferred_element_type=F32)
         + bn1[...])
    av = _silu(a)
    h_out[...] = jnp.dot(av, wn2[...], preferred_element_type=F32) + bn2[...]
    x_out[...] = x_ref[...] + gx_ref[...][:, :4]


def _tc_node(tbl, pos4, g0, g1, gx, wts):
    n = tbl.shape[0]
    rb = 1000
    const = lambda i: (0, 0)
    row = lambda i: (i, 0)
    wspecs = [pl.BlockSpec(w.shape, const) for w in wts]
    return pl.pallas_call(
        _node_body,
        grid=(n // rb,),
        in_specs=[pl.BlockSpec((rb, 128), row), pl.BlockSpec((rb, 4), row),
                  pl.BlockSpec((rb, 128), row), pl.BlockSpec((rb, 128), row),
                  pl.BlockSpec((rb, 128), row)] + wspecs,
        out_specs=[pl.BlockSpec((rb, 128), row), pl.BlockSpec((rb, 4), row)],
        out_shape=[jax.ShapeDtypeStruct((n, 128), F32),
                   jax.ShapeDtypeStruct((n, 4), F32)],
    )(tbl, pos4, g0, g1, gx, *wts)


def _reduce_body(t_ref, x_ref, hs_ref, xs_ref):
    sh = jnp.sum(t_ref[...], axis=0, keepdims=True)
    sx = jnp.sum(x_ref[...], axis=0, keepdims=True)

    @pl.when(pl.program_id(0) == 0)
    def _():
        hs_ref[...] = sh
        xs_ref[...] = sx

    @pl.when(pl.program_id(0) != 0)
    def _():
        hs_ref[...] += sh
        xs_ref[...] += sx


def _tc_reduce(tbl, pos4):
    n = tbl.shape[0]
    rb = 1000
    return pl.pallas_call(
        _reduce_body,
        grid=(n // rb,),
        in_specs=[pl.BlockSpec((rb, 128), lambda i: (i, 0)),
                  pl.BlockSpec((rb, 4), lambda i: (i, 0))],
        out_specs=[pl.BlockSpec((1, 128), lambda i: (0, 0)),
                   pl.BlockSpec((1, 4), lambda i: (0, 0))],
        out_shape=[jax.ShapeDtypeStruct((1, 128), F32),
                   jax.ShapeDtypeStruct((1, 4), F32)],
    )(tbl, pos4)


def _layer_weights(p):
    e1w = p["edge1"]["w"]
    edge = (e1w[:128], e1w[128:256], e1w[256:257],
            p["edge1"]["b"].reshape(1, -1),
            p["edge2"]["w"], p["edge2"]["b"].reshape(1, -1),
            p["coord1"]["w"], p["coord1"]["b"].reshape(1, -1),
            p["coord2"]["w"].reshape(1, -1))
    n1w = p["node1"]["w"]
    node = (n1w[:128], n1w[128:256], n1w[256:384],
            p["node1"]["b"].reshape(1, -1),
            p["node2"]["w"], p["node2"]["b"].reshape(1, -1))
    return edge, node


def kernel(h, pos, edge_index, params):
    n = h.shape[0]
    e = edge_index.shape[1]
    ep = -(-e // 10240) * 10240
    nacc = -(-(n + 1) // 640) * 640
    src = jnp.pad(edge_index[0], (0, ep - e))
    dst_g = jnp.pad(edge_index[1], (0, ep - e))
    dst2d = jnp.pad(edge_index[1], (0, ep - e),
                    constant_values=n).reshape(-1, 80)
    tbl = h
    pos4 = jnp.pad(pos, ((0, 0), (0, 1)))
    npad = -(-n // 128) * 128
    for name in ("layer1", "layer2", "layer3"):
        ewts, nwts = _layer_weights(params[name])
        dxf = _sc_dx(pos4.reshape(-1), src, dst_g)
        tblp = jnp.pad(tbl, ((0, npad - n), (0, 0)))
        ts, td = _sc_gather(tblp, src, dst_g)
        dx = dxf.reshape(-1, 4)
        m0, m1, msgp = _tc_edge(ts, td, dx, ewts)
        g0, g1, gx = _sc_scatter(m0, m1, msgp, dst2d, nacc)
        tbl, pos4 = _tc_node(tbl, pos4, g0, g1, gx, nwts)
    hsums, xsums = _tc_reduce(tbl, pos4)
    return jnp.concatenate([hsums, xsums[:, :3]], axis=1) / n


# speedup vs baseline: 3.6197x; 1.1754x over previous
import functools

import jax
import jax.numpy as jnp
from jax import lax
from jax.experimental import pallas as pl
from jax.experimental.pallas import tpu as pltpu
from jax.experimental.pallas import tpu_sc as plsc

NC = 2
NS = 16
F32 = jnp.float32


def _sc_dx(pos4, src, dst):
    n4 = pos4.shape[0]
    e = src.shape[0]
    nw = NC * NS
    per_w = e // nw
    ch = 512
    blk = 1024
    iters = per_w // ch
    outer = per_w // blk
    mesh = plsc.VectorSubcoreMesh(core_axis_name="c", subcore_axis_name="s")

    @functools.partial(
        pl.kernel,
        out_type=jax.ShapeDtypeStruct((e * 4,), F32),
        mesh=mesh,
        scratch_types=[
            pltpu.VMEM((blk,), jnp.int32),
            pltpu.VMEM((blk,), jnp.int32),
            pltpu.VMEM((2 * ch * 4,), F32),
            pltpu.VMEM((n4,), F32),
            pltpu.SemaphoreType.DMA,
        ],
        compiler_params=pltpu.CompilerParams(needs_layout_passes=False),
    )
    def k(p_hbm, src_hbm, dst_hbm, dx_hbm, si, di, dbuf, posv, sem_w):
        wid = lax.axis_index("s") * NC + lax.axis_index("c")
        base = wid * per_w
        lane = lax.iota(jnp.int32, 16)
        pltpu.sync_copy(p_hbm, posv)
        zv = jnp.zeros((16,), F32)
        for q in range(2 * ch * 4 // 16):
            dbuf[pl.ds(q * 16, 16)] = zv

        def wait_wb(i, b):
            off = base + i * ch
            pltpu.make_async_copy(dbuf.at[pl.ds(b * ch * 4, ch * 4)],
                                  dx_hbm.at[pl.ds(off * 4, ch * 4)],
                                  sem_w).wait()

        def step(i, u):
            b = u % 2
            off = base + i * ch

            @pl.when(i >= 2)
            def _():
                wait_wb(i - 2, b)

            for g in range(ch // 16):
                s16 = si[pl.ds(u * ch + g * 16, 16)]
                d16 = di[pl.ds(u * ch + g * 16, 16)]
                row4 = b * ch * 4 + (lane + g * 16) * 4
                for c in range(3):
                    cv = jnp.full((16,), c, jnp.int32)
                    xs = plsc.load_gather(posv, [s16 * 4 + cv])
                    xd = plsc.load_gather(posv, [d16 * 4 + cv])
                    plsc.store_scatter(dbuf, [row4 + cv], xs - xd)
            pltpu.async_copy(dbuf.at[pl.ds(b * ch * 4, ch * 4)],
                             dx_hbm.at[pl.ds(off * 4, ch * 4)], sem_w)

        def outer_body(g, carry):
            boff = base + g * blk
            pltpu.sync_copy(src_hbm.at[pl.ds(boff, blk)], si)
            pltpu.sync_copy(dst_hbm.at[pl.ds(boff, blk)], di)
            for u in range(blk // ch):
                step(g * (blk // ch) + u, u)
            return carry

        lax.fori_loop(0, outer, outer_body, 0)
        wait_wb(iters - 2, (iters - 2) % 2)
        wait_wb(iters - 1, (iters - 1) % 2)

    return k(pos4, src, dst)


def _sc_gather(tblp, src, dst):
    npad = tblp.shape[0]
    e = src.shape[0]
    nw = NC * NS
    per_w = e // nw
    ch = 80
    blk = 640
    iters = per_w // ch
    outer = per_w // blk
    rows_cp = npad // NS
    mesh = plsc.VectorSubcoreMesh(core_axis_name="c", subcore_axis_name="s")

    @functools.partial(
        pl.kernel,
        out_type=[jax.ShapeDtypeStruct((e, 128), F32),
                  jax.ShapeDtypeStruct((e, 128), F32)],
        mesh=mesh,
        scratch_types=[
            pltpu.VMEM((blk,), jnp.int32),
            pltpu.VMEM((blk,), jnp.int32),
            pltpu.VMEM((2 * ch, 128), F32),
            pltpu.VMEM((2 * ch, 128), F32),
            pltpu.VMEM_SHARED((npad, 128), F32),
            pltpu.SemaphoreType.DMA,
            pltpu.SemaphoreType.DMA,
        ],
        compiler_params=pltpu.CompilerParams(needs_layout_passes=False),
    )
    def k(t_hbm, src_hbm, dst_hbm, ts_hbm, td_hbm,
          si, di, rs, rd, t_sp, sem_g, sem_w):
        sid = lax.axis_index("s")
        wid = sid * NC + lax.axis_index("c")
        base = wid * per_w
        pltpu.sync_copy(t_hbm.at[pl.ds(sid * rows_cp, rows_cp)],
                        t_sp.at[pl.ds(sid * rows_cp, rows_cp)])
        plsc.subcore_barrier()

        def wait_wb(i, b):
            off = base + i * ch
            pltpu.make_async_copy(rs.at[pl.ds(b * ch, ch)],
                                  ts_hbm.at[pl.ds(off, ch)], sem_w).wait()
            pltpu.make_async_copy(rd.at[pl.ds(b * ch, ch)],
                                  td_hbm.at[pl.ds(off, ch)], sem_w).wait()

        def step(i, u):
            b = u % 2
            off = base + i * ch

            @pl.when(i >= 2)
            def _():
                wait_wb(i - 2, b)

            s1 = si.at[pl.ds(u * ch, ch)]
            d1 = di.at[pl.ds(u * ch, ch)]
            pltpu.async_copy(t_sp.at[s1], rs.at[pl.ds(b * ch, ch)], sem_g)
            pltpu.async_copy(t_sp.at[d1], rd.at[pl.ds(b * ch, ch)], sem_g)
            pltpu.make_async_copy(t_sp.at[s1],
                                  rs.at[pl.ds(b * ch, ch)], sem_g).wait()
            pltpu.make_async_copy(t_sp.at[d1],
                                  rd.at[pl.ds(b * ch, ch)], sem_g).wait()
            pltpu.async_copy(rs.at[pl.ds(b * ch, ch)],
                             ts_hbm.at[pl.ds(off, ch)], sem_w)
            pltpu.async_copy(rd.at[pl.ds(b * ch, ch)],
                             td_hbm.at[pl.ds(off, ch)], sem_w)

        def outer_body(g, carry):
            boff = base + g * blk
            pltpu.sync_copy(src_hbm.at[pl.ds(boff, blk)], si)
            pltpu.sync_copy(dst_hbm.at[pl.ds(boff, blk)], di)
            for u in range(blk // ch):
                step(g * (blk // ch) + u, u)
            return carry

        lax.fori_loop(0, outer, outer_body, 0)
        wait_wb(iters - 2, (iters - 2) % 2)
        wait_wb(iters - 1, (iters - 1) % 2)

    return k(tblp, src, dst)


def _sc_scatter(m0, m1, msgp, dst2d, nacc):
    sub = dst2d.shape[1]
    e = dst2d.shape[0] * sub
    rows_pt = nacc // NS
    zr = 32
    mesh = plsc.VectorSubcoreMesh(core_axis_name="c", subcore_axis_name="s")

    @functools.partial(
        pl.kernel,
        out_type=[jax.ShapeDtypeStruct((nacc, 128), F32),
                  jax.ShapeDtypeStruct((nacc, 128), F32),
                  jax.ShapeDtypeStruct((nacc, 128), F32),
                  jax.ShapeDtypeStruct((nacc, 128), F32)],
        mesh=mesh,
        scratch_types=[
            pltpu.VMEM((8, sub), jnp.int32),
            pltpu.VMEM((2 * sub, 128), F32),
            pltpu.VMEM((zr, 128), F32),
            pltpu.VMEM_SHARED((nacc, 128), F32),
            pltpu.SemaphoreType.DMA,
        ],
        compiler_params=pltpu.CompilerParams(needs_layout_passes=False),
    )
    def k(m0_hbm, m1_hbm, msg_hbm, idx_hbm, g0_hbm, g1_hbm, gxa_hbm, gxb_hbm,
          idxv, rbuf, zb, smem, sem_r):
        cid = lax.axis_index("c")
        sid = lax.axis_index("s")
        zv = jnp.zeros((16,), F32)
        for i in range(zr):
            for j in range(8):
                zb[i, pl.ds(j * 16, 16)] = zv
        r0 = sid * rows_pt

        def zero_acc():
            for q in range(rows_pt // zr):
                pltpu.sync_copy(zb, smem.at[pl.ds(r0 + q * zr, zr)])

        def accumulate(src_hbm, base_e, n_e):
            per_sub = n_e // NS
            units = per_sub // sub
            off0 = base_e + sid * per_sub
            pltpu.async_copy(src_hbm.at[pl.ds(off0, sub)],
                             rbuf.at[pl.ds(0, sub)], sem_r)

            def body(g, carry):
                goff = off0 + g * (8 * sub)
                irow = pl.multiple_of(goff // sub, 8)
                pltpu.sync_copy(idx_hbm.at[pl.ds(irow, 8)], idxv)
                for q in range(8):
                    u = g * 8 + q
                    b = q % 2
                    pltpu.make_async_copy(
                        src_hbm.at[pl.ds(off0 + u * sub, sub)],
                        rbuf.at[pl.ds(b * sub, sub)], sem_r).wait()

                    @pl.when(u + 1 < units)
                    def _():
                        pltpu.async_copy(
                            src_hbm.at[pl.ds(off0 + (u + 1) * sub, sub)],
                            rbuf.at[pl.ds(((q + 1) % 2) * sub, sub)], sem_r)

                    pltpu.sync_copy(rbuf.at[pl.ds(b * sub, sub)],
                                    smem.at[idxv.at[q]], add=True)
                return carry
            lax.fori_loop(0, units // 8, body, 0)

        def writeout(out_hbm):
            pltpu.sync_copy(smem.at[pl.ds(r0, rows_pt)],
                            out_hbm.at[pl.ds(r0, rows_pt)])

        half = e // 2
        zero_acc()
        plsc.subcore_barrier()

        @pl.when(cid == 0)
        def _():
            accumulate(m0_hbm, 0, e)

        @pl.when(cid == 1)
        def _():
            accumulate(m1_hbm, 0, e)

        plsc.subcore_barrier()

        @pl.when(cid == 0)
        def _():
            writeout(g0_hbm)

        @pl.when(cid == 1)
        def _():
            writeout(g1_hbm)

        plsc.subcore_barrier()
        zero_acc()
        plsc.subcore_barrier()

        @pl.when(cid == 0)
        def _():
            accumulate(msg_hbm, 0, half)

        @pl.when(cid == 1)
        def _():
            accumulate(msg_hbm, half, half)

        plsc.subcore_barrier()

        @pl.when(cid == 0)
        def _():
            writeout(gxa_hbm)

        @pl.when(cid == 1)
        def _():
            writeout(gxb_hbm)

    return k(m0, m1, msgp, dst2d)


def _silu(x):
    return x * jax.nn.sigmoid(x)


def _edge_body(ts_ref, td_ref, dx_ref, w1a, w1b, w1r, b1, w2, b2, wc1, bc1,
               wc2, m0_ref, m1_ref, msg_ref):
    hs = ts_ref[...]
    hd = td_ref[...]
    d = dx_ref[...]
    r = jnp.sum(d * d, axis=-1, keepdims=True)
    u = d / (jnp.sqrt(r) + 1e-30)
    a1 = (jnp.dot(hs, w1a[...], preferred_element_type=F32)
          + jnp.dot(hd, w1b[...], preferred_element_type=F32)
          + r * w1r[...] + b1[...])
    m1v = _silu(a1)
    a2 = jnp.dot(m1v, w2[...], preferred_element_type=F32) + b2[...]
    mv = _silu(a2)
    a3 = jnp.dot(mv, wc1[...], preferred_element_type=F32) + bc1[...]
    cv = _silu(a3)
    coef = jnp.sum(cv * wc2[...], axis=-1, keepdims=True)
    msg = coef * u
    m0_ref[...] = mv[:, :128]
    m1_ref[...] = mv[:, 128:]
    msg_ref[...] = jnp.concatenate(
        [msg, jnp.zeros((msg.shape[0], 124), F32)], axis=1)


def _tc_edge(ts, td, dx, wts):
    e = ts.shape[0]
    rb = 1280
    const = lambda i: (0, 0)
    row = lambda i: (i, 0)
    wspecs = [pl.BlockSpec(w.shape, const) for w in wts]
    return pl.pallas_call(
        _edge_body,
        grid=(e // rb,),
        in_specs=[pl.BlockSpec((rb, 128), row), pl.BlockSpec((rb, 128), row),
                  pl.BlockSpec((rb, 4), row)] + wspecs,
        out_specs=[pl.BlockSpec((rb, 128), row), pl.BlockSpec((rb, 128), row),
                   pl.BlockSpec((rb, 128), row)],
        out_shape=[jax.ShapeDtypeStruct((e, 128), F32),
                   jax.ShapeDtypeStruct((e, 128), F32),
                   jax.ShapeDtypeStruct((e, 128), F32)],
    )(ts, td, dx, *wts)


def _node_body(t_ref, x_ref, g0_ref, g1_ref, gxa_ref, gxb_ref, wn1a, wn1b0,
               wn1b1, bn1, wn2, bn2, h_out, x_out):
    a = (jnp.dot(t_ref[...], wn1a[...], preferred_element_type=F32)
         + jnp.dot(g0_ref[...], wn1b0[...], preferred_element_type=F32)
         + jnp.dot(g1_ref[...], wn1b1[...], preferred_element_type=F32)
         + bn1[...])
    av = _silu(a)
    h_out[...] = jnp.dot(av, wn2[...], preferred_element_type=F32) + bn2[...]
    x_out[...] = (x_ref[...] + gxa_ref[...][:, :4] + gxb_ref[...][:, :4])


def _tc_node(tbl, pos4, g0, g1, gxa, gxb, wts):
    n = tbl.shape[0]
    rb = 1000
    const = lambda i: (0, 0)
    row = lambda i: (i, 0)
    wspecs = [pl.BlockSpec(w.shape, const) for w in wts]
    return pl.pallas_call(
        _node_body,
        grid=(n // rb,),
        in_specs=[pl.BlockSpec((rb, 128), row), pl.BlockSpec((rb, 4), row),
                  pl.BlockSpec((rb, 128), row), pl.BlockSpec((rb, 128), row),
                  pl.BlockSpec((rb, 128), row), pl.BlockSpec((rb, 128), row)]
        + wspecs,
        out_specs=[pl.BlockSpec((rb, 128), row), pl.BlockSpec((rb, 4), row)],
        out_shape=[jax.ShapeDtypeStruct((n, 128), F32),
                   jax.ShapeDtypeStruct((n, 4), F32)],
    )(tbl, pos4, g0, g1, gxa, gxb, *wts)


def _reduce_body(t_ref, x_ref, hs_ref, xs_ref):
    sh = jnp.sum(t_ref[...], axis=0, keepdims=True)
    sx = jnp.sum(x_ref[...], axis=0, keepdims=True)

    @pl.when(pl.program_id(0) == 0)
    def _():
        hs_ref[...] = sh
        xs_ref[...] = sx

    @pl.when(pl.program_id(0) != 0)
    def _():
        hs_ref[...] += sh
        xs_ref[...] += sx


def _tc_reduce(tbl, pos4):
    n = tbl.shape[0]
    rb = 1000
    return pl.pallas_call(
        _reduce_body,
        grid=(n // rb,),
        in_specs=[pl.BlockSpec((rb, 128), lambda i: (i, 0)),
                  pl.BlockSpec((rb, 4), lambda i: (i, 0))],
        out_specs=[pl.BlockSpec((1, 128), lambda i: (0, 0)),
                   pl.BlockSpec((1, 4), lambda i: (0, 0))],
        out_shape=[jax.ShapeDtypeStruct((1, 128), F32),
                   jax.ShapeDtypeStruct((1, 4), F32)],
    )(tbl, pos4)


def _layer_weights(p):
    e1w = p["edge1"]["w"]
    edge = (e1w[:128], e1w[128:256], e1w[256:257],
            p["edge1"]["b"].reshape(1, -1),
            p["edge2"]["w"], p["edge2"]["b"].reshape(1, -1),
            p["coord1"]["w"], p["coord1"]["b"].reshape(1, -1),
            p["coord2"]["w"].reshape(1, -1))
    n1w = p["node1"]["w"]
    node = (n1w[:128], n1w[128:256], n1w[256:384],
            p["node1"]["b"].reshape(1, -1),
            p["node2"]["w"], p["node2"]["b"].reshape(1, -1))
    return edge, node


def kernel(h, pos, edge_index, params):
    n = h.shape[0]
    e = edge_index.shape[1]
    ep = -(-e // 10240) * 10240
    nacc = -(-(n + 1) // 640) * 640
    src = jnp.pad(edge_index[0], (0, ep - e))
    dst_g = jnp.pad(edge_index[1], (0, ep - e))
    dst2d = jnp.pad(edge_index[1], (0, ep - e),
                    constant_values=n).reshape(-1, 80)
    tbl = h
    pos4 = jnp.pad(pos, ((0, 0), (0, 1)))
    npad = -(-n // 128) * 128
    for name in ("layer1", "layer2", "layer3"):
        ewts, nwts = _layer_weights(params[name])
        dxf = _sc_dx(pos4.reshape(-1), src, dst_g)
        tblp = jnp.pad(tbl, ((0, npad - n), (0, 0)))
        ts, td = _sc_gather(tblp, src, dst_g)
        dx = dxf.reshape(-1, 4)
        m0, m1, msgp = _tc_edge(ts, td, dx, ewts)
        g0, g1, gxa, gxb = _sc_scatter(m0, m1, msgp, dst2d, nacc)
        tbl, pos4 = _tc_node(tbl, pos4, g0, g1, gxa, gxb, nwts)
    hsums, xsums = _tc_reduce(tbl, pos4)
    return jnp.concatenate([hsums, xsums[:, :3]], axis=1) / n


# speedup vs baseline: 3.6384x; 1.0052x over previous
import functools

import jax
import jax.numpy as jnp
from jax import lax
from jax.experimental import pallas as pl
from jax.experimental.pallas import tpu as pltpu
from jax.experimental.pallas import tpu_sc as plsc

NC = 2
NS = 16
F32 = jnp.float32


def _sc_dx(pos4, src, dst):
    n4 = pos4.shape[0]
    e = src.shape[0]
    nw = NC * NS
    per_w = e // nw
    ch = 512
    blk = 1024
    iters = per_w // ch
    outer = per_w // blk
    mesh = plsc.VectorSubcoreMesh(core_axis_name="c", subcore_axis_name="s")

    @functools.partial(
        pl.kernel,
        out_type=jax.ShapeDtypeStruct((e * 4,), F32),
        mesh=mesh,
        scratch_types=[
            pltpu.VMEM((blk,), jnp.int32),
            pltpu.VMEM((blk,), jnp.int32),
            pltpu.VMEM((2 * ch * 4,), F32),
            pltpu.VMEM((n4,), F32),
            pltpu.SemaphoreType.DMA,
        ],
        compiler_params=pltpu.CompilerParams(needs_layout_passes=False),
    )
    def k(p_hbm, src_hbm, dst_hbm, dx_hbm, si, di, dbuf, posv, sem_w):
        wid = lax.axis_index("s") * NC + lax.axis_index("c")
        base = wid * per_w
        lane = lax.iota(jnp.int32, 16)
        pltpu.sync_copy(p_hbm, posv)
        zv = jnp.zeros((16,), F32)
        for q in range(2 * ch * 4 // 16):
            dbuf[pl.ds(q * 16, 16)] = zv

        def wait_wb(i, b):
            off = base + i * ch
            pltpu.make_async_copy(dbuf.at[pl.ds(b * ch * 4, ch * 4)],
                                  dx_hbm.at[pl.ds(off * 4, ch * 4)],
                                  sem_w).wait()

        def step(i, u):
            b = u % 2
            off = base + i * ch

            @pl.when(i >= 2)
            def _():
                wait_wb(i - 2, b)

            for g in range(ch // 16):
                s16 = si[pl.ds(u * ch + g * 16, 16)]
                d16 = di[pl.ds(u * ch + g * 16, 16)]
                row4 = b * ch * 4 + (lane + g * 16) * 4
                for c in range(3):
                    cv = jnp.full((16,), c, jnp.int32)
                    xs = plsc.load_gather(posv, [s16 * 4 + cv])
                    xd = plsc.load_gather(posv, [d16 * 4 + cv])
                    plsc.store_scatter(dbuf, [row4 + cv], xs - xd)
            pltpu.async_copy(dbuf.at[pl.ds(b * ch * 4, ch * 4)],
                             dx_hbm.at[pl.ds(off * 4, ch * 4)], sem_w)

        def outer_body(g, carry):
            boff = base + g * blk
            pltpu.sync_copy(src_hbm.at[pl.ds(boff, blk)], si)
            pltpu.sync_copy(dst_hbm.at[pl.ds(boff, blk)], di)
            for u in range(blk // ch):
                step(g * (blk // ch) + u, u)
            return carry

        lax.fori_loop(0, outer, outer_body, 0)
        wait_wb(iters - 2, (iters - 2) % 2)
        wait_wb(iters - 1, (iters - 1) % 2)

    return k(pos4, src, dst)


def _sc_gather(tblp, src, dst):
    npad = tblp.shape[0]
    e = src.shape[0]
    nw = NC * NS
    per_w = e // nw
    ch = 80
    blk = 640
    iters = per_w // ch
    outer = per_w // blk
    rows_cp = npad // NS
    mesh = plsc.VectorSubcoreMesh(core_axis_name="c", subcore_axis_name="s")

    @functools.partial(
        pl.kernel,
        out_type=[jax.ShapeDtypeStruct((e, 128), F32),
                  jax.ShapeDtypeStruct((e, 128), F32)],
        mesh=mesh,
        scratch_types=[
            pltpu.VMEM((blk,), jnp.int32),
            pltpu.VMEM((blk,), jnp.int32),
            pltpu.VMEM((2 * ch, 128), F32),
            pltpu.VMEM((2 * ch, 128), F32),
            pltpu.VMEM_SHARED((npad, 128), F32),
            pltpu.SemaphoreType.DMA,
            pltpu.SemaphoreType.DMA,
        ],
        compiler_params=pltpu.CompilerParams(needs_layout_passes=False),
    )
    def k(t_hbm, src_hbm, dst_hbm, ts_hbm, td_hbm,
          si, di, rs, rd, t_sp, sem_g, sem_w):
        sid = lax.axis_index("s")
        wid = sid * NC + lax.axis_index("c")
        base = wid * per_w
        pltpu.sync_copy(t_hbm.at[pl.ds(sid * rows_cp, rows_cp)],
                        t_sp.at[pl.ds(sid * rows_cp, rows_cp)])
        plsc.subcore_barrier()

        def wait_wb(i, b):
            off = base + i * ch
            pltpu.make_async_copy(rs.at[pl.ds(b * ch, ch)],
                                  ts_hbm.at[pl.ds(off, ch)], sem_w).wait()
            pltpu.make_async_copy(rd.at[pl.ds(b * ch, ch)],
                                  td_hbm.at[pl.ds(off, ch)], sem_w).wait()

        def step(i, u):
            b = u % 2
            off = base + i * ch

            @pl.when(i >= 2)
            def _():
                wait_wb(i - 2, b)

            s1 = si.at[pl.ds(u * ch, ch)]
            d1 = di.at[pl.ds(u * ch, ch)]
            pltpu.async_copy(t_sp.at[s1], rs.at[pl.ds(b * ch, ch)], sem_g)
            pltpu.async_copy(t_sp.at[d1], rd.at[pl.ds(b * ch, ch)], sem_g)
            pltpu.make_async_copy(t_sp.at[s1],
                                  rs.at[pl.ds(b * ch, ch)], sem_g).wait()
            pltpu.make_async_copy(t_sp.at[d1],
                                  rd.at[pl.ds(b * ch, ch)], sem_g).wait()
            pltpu.async_copy(rs.at[pl.ds(b * ch, ch)],
                             ts_hbm.at[pl.ds(off, ch)], sem_w)
            pltpu.async_copy(rd.at[pl.ds(b * ch, ch)],
                             td_hbm.at[pl.ds(off, ch)], sem_w)

        def outer_body(g, carry):
            boff = base + g * blk
            pltpu.sync_copy(src_hbm.at[pl.ds(boff, blk)], si)
            pltpu.sync_copy(dst_hbm.at[pl.ds(boff, blk)], di)
            for u in range(blk // ch):
                step(g * (blk // ch) + u, u)
            return carry

        lax.fori_loop(0, outer, outer_body, 0)
        wait_wb(iters - 2, (iters - 2) % 2)
        wait_wb(iters - 1, (iters - 1) % 2)

    return k(tblp, src, dst)


def _sc_scatter(m0, m1, msgp, dst2d, nacc):
    sub = dst2d.shape[1]
    e = dst2d.shape[0] * sub
    rows_pt = nacc // NS
    zr = 32
    mesh = plsc.VectorSubcoreMesh(core_axis_name="c", subcore_axis_name="s")

    @functools.partial(
        pl.kernel,
        out_type=[jax.ShapeDtypeStruct((nacc, 128), F32),
                  jax.ShapeDtypeStruct((nacc, 128), F32),
                  jax.ShapeDtypeStruct((nacc, 128), F32),
                  jax.ShapeDtypeStruct((nacc, 128), F32)],
        mesh=mesh,
        scratch_types=[
            pltpu.VMEM((8, sub), jnp.int32),
            pltpu.VMEM((2 * sub, 128), F32),
            pltpu.VMEM((zr, 128), F32),
            pltpu.VMEM_SHARED((nacc, 128), F32),
            pltpu.SemaphoreType.DMA,
        ],
        compiler_params=pltpu.CompilerParams(needs_layout_passes=False),
    )
    def k(m0_hbm, m1_hbm, msg_hbm, idx_hbm, g0_hbm, g1_hbm, gxa_hbm, gxb_hbm,
          idxv, rbuf, zb, smem, sem_r):
        cid = lax.axis_index("c")
        sid = lax.axis_index("s")
        zv = jnp.zeros((16,), F32)
        for i in range(zr):
            for j in range(8):
                zb[i, pl.ds(j * 16, 16)] = zv
        r0 = sid * rows_pt

        def zero_acc():
            for q in range(rows_pt // zr):
                pltpu.sync_copy(zb, smem.at[pl.ds(r0 + q * zr, zr)])

        def accumulate(src_hbm, base_e, n_e):
            per_sub = n_e // NS
            units = per_sub // sub
            off0 = base_e + sid * per_sub
            pltpu.async_copy(src_hbm.at[pl.ds(off0, sub)],
                             rbuf.at[pl.ds(0, sub)], sem_r)

            def body(g, carry):
                goff = off0 + g * (8 * sub)
                irow = pl.multiple_of(goff // sub, 8)
                pltpu.sync_copy(idx_hbm.at[pl.ds(irow, 8)], idxv)
                for q in range(8):
                    u = g * 8 + q
                    b = q % 2
                    pltpu.make_async_copy(
                        src_hbm.at[pl.ds(off0 + u * sub, sub)],
                        rbuf.at[pl.ds(b * sub, sub)], sem_r).wait()

                    @pl.when(u + 1 < units)
                    def _():
                        pltpu.async_copy(
                            src_hbm.at[pl.ds(off0 + (u + 1) * sub, sub)],
                            rbuf.at[pl.ds(((q + 1) % 2) * sub, sub)], sem_r)

                    pltpu.sync_copy(rbuf.at[pl.ds(b * sub, sub)],
                                    smem.at[idxv.at[q]], add=True)
                return carry
            lax.fori_loop(0, units // 8, body, 0)

        def writeout(out_hbm):
            pltpu.sync_copy(smem.at[pl.ds(r0, rows_pt)],
                            out_hbm.at[pl.ds(r0, rows_pt)])

        half = e // 2
        zero_acc()
        plsc.subcore_barrier()

        @pl.when(cid == 0)
        def _():
            accumulate(m0_hbm, 0, e)

        @pl.when(cid == 1)
        def _():
            accumulate(m1_hbm, 0, e)

        plsc.subcore_barrier()

        @pl.when(cid == 0)
        def _():
            writeout(g0_hbm)

        @pl.when(cid == 1)
        def _():
            writeout(g1_hbm)

        plsc.subcore_barrier()
        zero_acc()
        plsc.subcore_barrier()

        @pl.when(cid == 0)
        def _():
            accumulate(msg_hbm, 0, half)

        @pl.when(cid == 1)
        def _():
            accumulate(msg_hbm, half, half)

        plsc.subcore_barrier()

        @pl.when(cid == 0)
        def _():
            writeout(gxa_hbm)

        @pl.when(cid == 1)
        def _():
            writeout(gxb_hbm)

    return k(m0, m1, msgp, dst2d)


def _silu(x):
    return x * jax.nn.sigmoid(x)


def _edge_body(ts_ref, td_ref, dx_ref, w1a, w1b, w1r, b1, w2, b2, wc1, bc1,
               wc2, m0_ref, m1_ref, msg_ref):
    hs = ts_ref[...]
    hd = td_ref[...]
    d = dx_ref[...]
    r = jnp.sum(d * d, axis=-1, keepdims=True)
    u = d / (jnp.sqrt(r) + 1e-30)
    a1 = (jnp.dot(hs, w1a[...], preferred_element_type=F32)
          + jnp.dot(hd, w1b[...], preferred_element_type=F32)
          + r * w1r[...] + b1[...])
    m1v = _silu(a1)
    a2 = jnp.dot(m1v.astype(jnp.bfloat16), w2[...],
                 preferred_element_type=F32) + b2[...]
    mv = _silu(a2)
    a3 = jnp.dot(mv.astype(jnp.bfloat16), wc1[...],
                 preferred_element_type=F32) + bc1[...]
    cv = _silu(a3)
    coef = jnp.sum(cv * wc2[...], axis=-1, keepdims=True)
    msg = coef * u
    m0_ref[...] = mv[:, :128]
    m1_ref[...] = mv[:, 128:]
    msg_ref[...] = jnp.concatenate(
        [msg, jnp.zeros((msg.shape[0], 124), F32)], axis=1)


def _tc_edge(ts, td, dx, wts):
    e = ts.shape[0]
    rb = 1280
    const = lambda i: (0, 0)
    row = lambda i: (i, 0)
    wspecs = [pl.BlockSpec(w.shape, const) for w in wts]
    return pl.pallas_call(
        _edge_body,
        grid=(e // rb,),
        in_specs=[pl.BlockSpec((rb, 128), row), pl.BlockSpec((rb, 128), row),
                  pl.BlockSpec((rb, 4), row)] + wspecs,
        out_specs=[pl.BlockSpec((rb, 128), row), pl.BlockSpec((rb, 128), row),
                   pl.BlockSpec((rb, 128), row)],
        out_shape=[jax.ShapeDtypeStruct((e, 128), F32),
                   jax.ShapeDtypeStruct((e, 128), F32),
                   jax.ShapeDtypeStruct((e, 128), F32)],
    )(ts, td, dx, *wts)


def _node_body(t_ref, x_ref, g0_ref, g1_ref, gxa_ref, gxb_ref, wn1a, wn1b0,
               wn1b1, bn1, wn2, bn2, h_out, x_out):
    a = (jnp.dot(t_ref[...], wn1a[...], preferred_element_type=F32)
         + jnp.dot(g0_ref[...], wn1b0[...], preferred_element_type=F32)
         + jnp.dot(g1_ref[...], wn1b1[...], preferred_element_type=F32)
         + bn1[...])
    av = _silu(a)
    h_out[...] = jnp.dot(av, wn2[...], preferred_element_type=F32) + bn2[...]
    x_out[...] = (x_ref[...] + gxa_ref[...][:, :4] + gxb_ref[...][:, :4])


def _tc_node(tbl, pos4, g0, g1, gxa, gxb, wts):
    n = tbl.shape[0]
    rb = 1000
    const = lambda i: (0, 0)
    row = lambda i: (i, 0)
    wspecs = [pl.BlockSpec(w.shape, const) for w in wts]
    return pl.pallas_call(
        _node_body,
        grid=(n // rb,),
        in_specs=[pl.BlockSpec((rb, 128), row), pl.BlockSpec((rb, 4), row),
                  pl.BlockSpec((rb, 128), row), pl.BlockSpec((rb, 128), row),
                  pl.BlockSpec((rb, 128), row), pl.BlockSpec((rb, 128), row)]
        + wspecs,
        out_specs=[pl.BlockSpec((rb, 128), row), pl.BlockSpec((rb, 4), row)],
        out_shape=[jax.ShapeDtypeStruct((n, 128), F32),
                   jax.ShapeDtypeStruct((n, 4), F32)],
    )(tbl, pos4, g0, g1, gxa, gxb, *wts)


def _reduce_body(t_ref, x_ref, hs_ref, xs_ref):
    sh = jnp.sum(t_ref[...], axis=0, keepdims=True)
    sx = jnp.sum(x_ref[...], axis=0, keepdims=True)

    @pl.when(pl.program_id(0) == 0)
    def _():
        hs_ref[...] = sh
        xs_ref[...] = sx

    @pl.when(pl.program_id(0) != 0)
    def _():
        hs_ref[...] += sh
        xs_ref[...] += sx


def _tc_reduce(tbl, pos4):
    n = tbl.shape[0]
    rb = 1000
    return pl.pallas_call(
        _reduce_body,
        grid=(n // rb,),
        in_specs=[pl.BlockSpec((rb, 128), lambda i: (i, 0)),
                  pl.BlockSpec((rb, 4), lambda i: (i, 0))],
        out_specs=[pl.BlockSpec((1, 128), lambda i: (0, 0)),
                   pl.BlockSpec((1, 4), lambda i: (0, 0))],
        out_shape=[jax.ShapeDtypeStruct((1, 128), F32),
                   jax.ShapeDtypeStruct((1, 4), F32)],
    )(tbl, pos4)


def _layer_weights(p):
    e1w = p["edge1"]["w"]
    edge = (e1w[:128], e1w[128:256], e1w[256:257],
            p["edge1"]["b"].reshape(1, -1),
            p["edge2"]["w"].astype(jnp.bfloat16),
            p["edge2"]["b"].reshape(1, -1),
            p["coord1"]["w"].astype(jnp.bfloat16),
            p["coord1"]["b"].reshape(1, -1),
            p["coord2"]["w"].reshape(1, -1))
    n1w = p["node1"]["w"]
    node = (n1w[:128], n1w[128:256], n1w[256:384],
            p["node1"]["b"].reshape(1, -1),
            p["node2"]["w"], p["node2"]["b"].reshape(1, -1))
    return edge, node


def kernel(h, pos, edge_index, params):
    n = h.shape[0]
    e = edge_index.shape[1]
    ep = -(-e // 10240) * 10240
    nacc = -(-(n + 1) // 640) * 640
    src = jnp.pad(edge_index[0], (0, ep - e))
    dst_g = jnp.pad(edge_index[1], (0, ep - e))
    dst2d = jnp.pad(edge_index[1], (0, ep - e),
                    constant_values=n).reshape(-1, 80)
    tbl = h
    pos4 = jnp.pad(pos, ((0, 0), (0, 1)))
    npad = -(-n // 128) * 128
    for name in ("layer1", "layer2", "layer3"):
        ewts, nwts = _layer_weights(params[name])
        dxf = _sc_dx(pos4.reshape(-1), src, dst_g)
        tblp = jnp.pad(tbl, ((0, npad - n), (0, 0)))
        ts, td = _sc_gather(tblp, src, dst_g)
        dx = dxf.reshape(-1, 4)
        m0, m1, msgp = _tc_edge(ts, td, dx, ewts)
        g0, g1, gxa, gxb = _sc_scatter(m0, m1, msgp, dst2d, nacc)
        tbl, pos4 = _tc_node(tbl, pos4, g0, g1, gxa, gxb, nwts)
    hsums, xsums = _tc_reduce(tbl, pos4)
    return jnp.concatenate([hsums, xsums[:, :3]], axis=1) / n
